# Initial kernel scaffold; baseline (speedup 1.0000x reference)
import jax
import jax.numpy as jnp
from jax.experimental import pallas as pl


def kernel(q_sub, q_rel, hidden, edges, nodes, old_nodes_new_idx, batchsize, rela_embed, Ws, Wr, Wqr, b_qr, w_alpha, b_alpha, W_h):
    raise NotImplementedError("write your pallas kernel here")


# speedup vs baseline: 1.2006x; 1.2006x over previous
import jax
import jax.numpy as jnp
from jax import lax
from jax.experimental import pallas as pl
from jax.experimental.pallas import tpu as pltpu
from jax.experimental.pallas import tpu_sc as plsc

NC, NS, LANES = 2, 16, 16
NW = NC * NS
K = 64
D = 256
DH = 128


def _prep_body(hid_ref, rel_ref, wsT, wrT, wqrT, bqr, a_ref, b_ref, c_ref):
    h = hid_ref[...]
    r = rel_ref[...]
    a_ref[...] = jnp.dot(h, wsT[...], preferred_element_type=jnp.float32) + bqr[...]
    b_ref[...] = jnp.dot(r, wrT[...], preferred_element_type=jnp.float32)
    c_ref[...] = jnp.dot(r, wqrT[...], preferred_element_type=jnp.float32)


def _final_body(ph0_ref, ph1_ref, whT1, whT2, out_ref):
    a = ph0_ref[0] + ph0_ref[1]
    b = ph1_ref[0] + ph1_ref[1]
    out_ref[...] = (jnp.dot(a, whT1[...], preferred_element_type=jnp.float32)
                    + jnp.dot(b, whT2[...], preferred_element_type=jnp.float32))


def _make_alpha_body(n_edge, ept):
    nchunk = ept // K

    def body(suba, rela_i, ridxa, qrel_h, a_h, b_h, c_h, wal_h, bal_h,
             alpha_out,
             qrel_v, wal_v, bal_v, sub_v, rel_v, ridx_v, cidx_v,
             accbuf, ta_v, tb_v, tc_v, alpha_v, sem):
        cid = lax.axis_index("c")
        sid = lax.axis_index("s")
        w = cid * NS + sid

        pltpu.sync_copy(qrel_h, qrel_v)
        pltpu.sync_copy(wal_h, wal_v)
        pltpu.sync_copy(bal_h, bal_v)
        iot = lax.iota(jnp.int32, 16)

        def chunk_body(g, carry):
            base = pl.multiple_of(w * ept + g * K, 8)
            pltpu.sync_copy(suba.at[pl.ds(base, K)], sub_v)
            pltpu.sync_copy(rela_i.at[pl.ds(base, K)], rel_v)
            pltpu.sync_copy(ridxa.at[pl.ds(base, K)], ridx_v)
            for t in range(K // 16):
                rv = ridx_v[pl.ds(t * 16, 16)]
                cidx_v[pl.ds(t * 16, 16)] = plsc.load_gather(qrel_v, [rv])
            cps = [pltpu.async_copy(a_h.at[sub_v], ta_v, sem),
                   pltpu.async_copy(b_h.at[rel_v], tb_v, sem),
                   pltpu.async_copy(c_h.at[cidx_v], tc_v, sem)]
            for c in cps:
                c.wait()

            def group_alpha(t, carry2):
                def edge_acc(e, carry3):
                    i = t * 16 + e

                    def feat(j, acc):
                        sl = pl.ds(j * 16, 16)
                        pre = ta_v[i, sl] + tb_v[i, sl] + tc_v[i, sl]
                        return acc + jnp.maximum(pre, 0.0) * wal_v[sl]
                    acc = lax.fori_loop(0, 16, feat, jnp.zeros((16,), jnp.float32))
                    accbuf[e, :] = acc
                    return carry3
                lax.fori_loop(0, 16, edge_acc, 0)

                def colsum(j, s):
                    return s + plsc.load_gather(
                        accbuf, [iot, jnp.full((16,), j, jnp.int32)])
                s = lax.fori_loop(0, 16, colsum, jnp.zeros((16,), jnp.float32))
                av = 1.0 / (1.0 + jnp.exp(-(s + bal_v[...])))
                eid = (base + t * 16) + iot
                av = jnp.where(eid < n_edge, av, 0.0)
                alpha_v[pl.ds(g * K + t * 16, 16)] = av
                return carry2
            lax.fori_loop(0, K // 16, group_alpha, 0)
            return carry
        lax.fori_loop(0, nchunk, chunk_body, 0)
        pltpu.sync_copy(alpha_v, alpha_out.at[pl.ds(w * ept, ept)])

    return body


def _make_agg_body(ept, npad):
    nchunk = ept // K
    rows_per_tile = npad // NS
    zr = 128

    def body(suba, rela_i, obja, alpha_h, hm1_h, rm1_h, hm2_h, rm2_h,
             ph0, ph1,
             agg, sub_v, rel_v, obj_v, alpha_v, hm_v, rm_v, msg_v, zbuf, sem):
        cid = lax.axis_index("c")
        sid = lax.axis_index("s")
        w = cid * NS + sid
        row0 = sid * rows_per_tile

        def zrow(r, carry):
            for j in range(8):
                zbuf[r, pl.ds(j * 16, 16)] = jnp.zeros((16,), jnp.float32)
            return carry
        lax.fori_loop(0, zr, zrow, 0)

        def zero_agg():
            for q in range(rows_per_tile // zr):
                pltpu.sync_copy(zbuf, agg.at[pl.ds(row0 + q * zr, zr)])

        def run_pass(hm_h, rm_h, pout):
            def chunk_body(g, carry):
                base = pl.multiple_of(w * ept + g * K, 8)
                pltpu.sync_copy(suba.at[pl.ds(base, K)], sub_v)
                pltpu.sync_copy(rela_i.at[pl.ds(base, K)], rel_v)
                pltpu.sync_copy(obja.at[pl.ds(base, K)], obj_v)
                pltpu.sync_copy(alpha_h.at[pl.ds(base, K)], alpha_v)
                cps = [pltpu.async_copy(hm_h.at[sub_v], hm_v, sem),
                       pltpu.async_copy(rm_h.at[rel_v], rm_v, sem)]
                for c in cps:
                    c.wait()

                def edge_msg(i, carry2):
                    a = plsc.load_gather(alpha_v, [jnp.full((16,), i, jnp.int32)])

                    def feat(j, carry3):
                        sl = pl.ds(j * 16, 16)
                        msg_v[i, sl] = a * (hm_v[i, sl] + rm_v[i, sl])
                        return carry3
                    lax.fori_loop(0, 8, feat, 0)
                    return carry2
                lax.fori_loop(0, K, edge_msg, 0)
                pltpu.sync_copy(msg_v, agg.at[obj_v], add=True)
                return carry
            lax.fori_loop(0, nchunk, chunk_body, 0)
            plsc.subcore_barrier()
            pltpu.sync_copy(agg.at[pl.ds(row0, rows_per_tile)],
                            pout.at[cid, pl.ds(row0, rows_per_tile)])

        zero_agg()
        plsc.subcore_barrier()
        run_pass(hm1_h, rm1_h, ph0)
        zero_agg()
        plsc.subcore_barrier()
        run_pass(hm2_h, rm2_h, ph1)

    return body


def kernel(q_sub, q_rel, hidden, edges, nodes, old_nodes_new_idx, batchsize,
           rela_embed, Ws, Wr, Wqr, b_qr, w_alpha, b_alpha, W_h):
    n_node = nodes.shape[0]
    n_edge = edges.shape[0]
    n_rel = rela_embed.shape[0]
    f32 = jnp.float32

    sub = edges[:, 4].astype(jnp.int32)
    rel = edges[:, 2].astype(jnp.int32)
    obj = edges[:, 5].astype(jnp.int32)
    ridx = edges[:, 0].astype(jnp.int32)

    npad = ((max(n_node, n_rel) + 255) // 256) * 256
    ept = ((n_edge + NW * K - 1) // (NW * K)) * K
    epad = ept * NW
    pad = epad - n_edge
    suba = jnp.pad(sub, (0, pad))
    rela_i = jnp.pad(rel, (0, pad))
    obja = jnp.pad(obj, (0, pad))
    ridxa = jnp.pad(ridx, (0, pad))

    hid_p = jnp.pad(hidden.astype(f32), ((0, npad - n_node), (0, 0)))
    rel_p = jnp.pad(rela_embed.astype(f32), ((0, npad - n_rel), (0, 0)))

    nblk = npad // 256
    tbl_a, tbl_b, tbl_c = pl.pallas_call(
        _prep_body,
        grid=(nblk,),
        in_specs=[
            pl.BlockSpec((256, D), lambda i: (i, 0)),
            pl.BlockSpec((256, D), lambda i: (i, 0)),
            pl.BlockSpec((D, D), lambda i: (0, 0)),
            pl.BlockSpec((D, D), lambda i: (0, 0)),
            pl.BlockSpec((D, D), lambda i: (0, 0)),
            pl.BlockSpec((1, D), lambda i: (0, 0)),
        ],
        out_specs=[
            pl.BlockSpec((256, D), lambda i: (i, 0)),
            pl.BlockSpec((256, D), lambda i: (i, 0)),
            pl.BlockSpec((256, D), lambda i: (i, 0)),
        ],
        out_shape=[jax.ShapeDtypeStruct((npad, D), f32)] * 3,
    )(hid_p, rel_p, Ws.T.astype(f32), Wr.T.astype(f32), Wqr.T.astype(f32),
      b_qr.reshape(1, D).astype(f32))

    wal = w_alpha.reshape(-1).astype(f32)
    bal = jnp.broadcast_to(b_alpha.astype(f32), (16,))

    mesh = plsc.VectorSubcoreMesh(core_axis_name="c", subcore_axis_name="s",
                                  num_cores=NC, num_subcores=NS)
    sc_params = pltpu.CompilerParams(needs_layout_passes=False)

    alpha_fn = pl.kernel(
        _make_alpha_body(n_edge, ept),
        out_type=jax.ShapeDtypeStruct((epad,), f32),
        mesh=mesh,
        compiler_params=sc_params,
        scratch_types=[
            pltpu.VMEM((q_rel.shape[0],), jnp.int32),
            pltpu.VMEM((D,), f32),
            pltpu.VMEM((16,), f32),
            pltpu.VMEM((K,), jnp.int32),
            pltpu.VMEM((K,), jnp.int32),
            pltpu.VMEM((K,), jnp.int32),
            pltpu.VMEM((K,), jnp.int32),
            pltpu.VMEM((16, 16), f32),
            pltpu.VMEM((K, D), f32),
            pltpu.VMEM((K, D), f32),
            pltpu.VMEM((K, D), f32),
            pltpu.VMEM((ept,), f32),
            pltpu.SemaphoreType.DMA,
        ],
    )
    alphas = alpha_fn(suba, rela_i, ridxa, q_rel.astype(jnp.int32),
                      tbl_a, tbl_b, tbl_c, wal, bal)

    hm1 = hidden[:, :DH].astype(f32)
    hm2 = hidden[:, DH:].astype(f32)
    rm1 = rela_embed[:, :DH].astype(f32)
    rm2 = rela_embed[:, DH:].astype(f32)

    agg_fn = pl.kernel(
        _make_agg_body(ept, npad),
        out_type=(jax.ShapeDtypeStruct((NC, npad, DH), f32),
                  jax.ShapeDtypeStruct((NC, npad, DH), f32)),
        mesh=mesh,
        compiler_params=sc_params,
        scratch_types=[
            pltpu.VMEM_SHARED((npad, DH), f32),
            pltpu.VMEM((K,), jnp.int32),
            pltpu.VMEM((K,), jnp.int32),
            pltpu.VMEM((K,), jnp.int32),
            pltpu.VMEM((K,), f32),
            pltpu.VMEM((K, DH), f32),
            pltpu.VMEM((K, DH), f32),
            pltpu.VMEM((K, DH), f32),
            pltpu.VMEM((128, DH), f32),
            pltpu.SemaphoreType.DMA,
        ],
    )
    ph0, ph1 = agg_fn(suba, rela_i, obja, alphas, hm1, rm1, hm2, rm2)

    whT = W_h.T.astype(f32)
    out = pl.pallas_call(
        _final_body,
        grid=(nblk,),
        in_specs=[
            pl.BlockSpec((NC, 256, DH), lambda i: (0, i, 0)),
            pl.BlockSpec((NC, 256, DH), lambda i: (0, i, 0)),
            pl.BlockSpec((DH, D), lambda i: (0, 0)),
            pl.BlockSpec((DH, D), lambda i: (0, 0)),
        ],
        out_specs=pl.BlockSpec((256, D), lambda i: (i, 0)),
        out_shape=jax.ShapeDtypeStruct((npad, D), f32),
    )(ph0, ph1, whT[:DH], whT[DH:])

    return out[:n_node]
